# baseline (speedup 1.0000x reference)
import functools

import jax
import jax.numpy as jnp
from jax import lax
from jax.experimental import pallas as pl
from jax.experimental.pallas import tpu as pltpu
from jax.experimental.pallas import tpu_sc as plsc

DIM = 128
HALF = 64
NXY = 1024
INV_D = 1024.0
N_POINTS = 1024 * 200
NW = 32
CHUNK = 128
PER_W = N_POINTS // NW
N_CHUNKS = PER_W // CHUNK


def _sc_body(t_hbm, w_hbm, b_hbm, table_hbm, out_hbm,
             tbuf, idxbuf, gbuf, stage, wbuf, bbuf, sem, wsem):
    wid = lax.axis_index("s") * 2 + lax.axis_index("c")

    pltpu.sync_copy(w_hbm, wbuf)
    pltpu.sync_copy(b_hbm, bbuf)
    w0 = [wbuf[pl.ds(k * 16, 16)] for k in range(4)]
    w1 = [wbuf[pl.ds(64 + k * 16, 16)] for k in range(4)]
    bb = [bbuf[pl.ds(k * 16, 16)] for k in range(4)]
    lanes = lax.broadcasted_iota(jnp.int32, (16,), 0)

    def chunk_body(c, _):
        base = (wid * N_CHUNKS + c) * CHUNK
        pltpu.sync_copy(t_hbm.at[pl.ds(base * 2, CHUNK * 2)], tbuf)

        for g in range(CHUNK // 16):
            xi = jnp.int32(g * 32) + 2 * lanes
            xv = plsc.load_gather(tbuf, [xi])
            yv = plsc.load_gather(tbuf, [xi + 1])
            ix = jnp.clip((xv * INV_D).astype(jnp.int32), 0, NXY - 1)
            iy = jnp.clip((yv * INV_D).astype(jnp.int32), 0, NXY - 1)
            idxbuf[pl.ds(g * 16, 16)] = ix * NXY + iy

        gather = pltpu.make_async_copy(table_hbm.at[idxbuf], gbuf, sem)
        gather.start()

        def lin_body(p, _):
            xb = plsc.load_gather(tbuf, [jnp.full((16,), 2 * p, jnp.int32)])
            yb = plsc.load_gather(tbuf, [jnp.full((16,), 2 * p + 1, jnp.int32)])
            for k in range(4):
                stage[p, pl.ds(k * 16, 16)] = xb * w0[k] + yb * w1[k] + bb[k]
            return _

        lax.fori_loop(0, CHUNK, lin_body, None, unroll=2)

        gather.wait()

        def mv_body(p, _):
            for k in range(4):
                stage[p, pl.ds(HALF + k * 16, 16)] = gbuf[p, pl.ds(k * 16, 16)]
            return _

        lax.fori_loop(0, CHUNK, mv_body, None, unroll=2)
        pltpu.sync_copy(stage, out_hbm.at[pl.ds(base, CHUNK), :])
        return _

    lax.fori_loop(0, N_CHUNKS, chunk_body, None)


@jax.jit
def kernel(T, W1, b1, grid_table):
    mesh = plsc.VectorSubcoreMesh(core_axis_name="c", subcore_axis_name="s")
    run = pl.kernel(
        _sc_body,
        out_type=jax.ShapeDtypeStruct((N_POINTS, DIM), jnp.float32),
        mesh=mesh,
        scratch_types=[
            pltpu.VMEM((CHUNK * 2,), jnp.float32),
            pltpu.VMEM((CHUNK,), jnp.int32),
            pltpu.VMEM((CHUNK, DIM), jnp.float32),
            pltpu.VMEM((CHUNK, DIM), jnp.float32),
            pltpu.VMEM((2 * HALF,), jnp.float32),
            pltpu.VMEM((HALF,), jnp.float32),
            pltpu.SemaphoreType.DMA,
            pltpu.SemaphoreType.DMA,
        ],
        compiler_params=pltpu.CompilerParams(
            needs_layout_passes=False, use_tc_tiling_on_sc=True),
    )
    table128 = jnp.pad(grid_table, ((0, 0), (0, DIM - HALF)))
    out = run(T.reshape(-1), W1.reshape(-1), b1, table128)
    return out.reshape(T.shape[0], T.shape[1], DIM)
